# baseline (speedup 1.0000x reference)
import jax
import jax.numpy as jnp
from jax.experimental import pallas as pl
from jax.experimental.pallas import tpu as pltpu

T, D, H, E, TOPK = 256, 1024, 512, 64, 2
S = T * TOPK
TILE = 8
NT = 120
NTE = 128
ROWS = 1024
F32 = jnp.float32


def _routing_body(eflat_ref, pflat_ref, pos_ref, row_tok_ref, prob2d_ref,
                  te_ref):
    eflat = eflat_ref[...]
    pflat = pflat_ref[...]
    iota_e = jax.lax.broadcasted_iota(jnp.int32, (1, E), 1)
    onehot = (eflat == iota_e).astype(F32)
    r_i = jax.lax.broadcasted_iota(jnp.int32, (S, S), 0)
    c_i = jax.lax.broadcasted_iota(jnp.int32, (S, S), 1)
    lts = (r_i >= c_i).astype(F32)
    cum = jnp.dot(lts, onehot, preferred_element_type=F32, precision=jax.lax.Precision.HIGHEST)
    rank = jnp.sum(onehot * (cum - 1.0), axis=1, keepdims=True)
    counts = jnp.sum(onehot, axis=0, keepdims=True)
    padded = jnp.floor((counts + (TILE - 1)) / TILE) * TILE
    r64 = jax.lax.broadcasted_iota(jnp.int32, (E, E), 0)
    c64 = jax.lax.broadcasted_iota(jnp.int32, (E, E), 1)
    mstrict = (r64 < c64).astype(F32)
    start = jnp.dot(padded, mstrict, preferred_element_type=F32, precision=jax.lax.Precision.HIGHEST)
    pos_f = jnp.sum(onehot * start, axis=1, keepdims=True) + rank
    eye = (r_i == c_i).astype(F32)
    pos_t = jnp.dot(jnp.ones((1, S), F32), eye * pos_f,
                    preferred_element_type=F32, precision=jax.lax.Precision.HIGHEST)
    rr = jax.lax.broadcasted_iota(jnp.int32, (ROWS, 1), 0).astype(F32)
    perm = (rr == pos_t).astype(F32)
    tok = (jax.lax.broadcasted_iota(jnp.int32, (S, 1), 0) // TOPK).astype(F32)
    row_tok = jnp.dot(perm, tok, preferred_element_type=F32, precision=jax.lax.Precision.HIGHEST)
    row_prob = jnp.dot(perm, pflat, preferred_element_type=F32, precision=jax.lax.Precision.HIGHEST)
    ti = jax.lax.broadcasted_iota(jnp.int32, (NTE, 1), 0).astype(F32) * TILE
    te_hit = ((ti >= start) & (ti < start + padded)).astype(F32)
    tile_expert = jnp.sum(te_hit * iota_e.astype(F32), axis=1, keepdims=True)

    pos_ref[...] = pos_f.astype(jnp.int32)
    row_tok_ref[...] = row_tok.astype(jnp.int32)
    prob2d_ref[...] = jnp.broadcast_to(row_prob, (ROWS, 128))
    te_ref[...] = tile_expert.astype(jnp.int32)


def _routing(eflat, pflat):
    return pl.pallas_call(
        _routing_body,
        out_shape=(
            jax.ShapeDtypeStruct((S, 1), jnp.int32),
            jax.ShapeDtypeStruct((ROWS, 1), jnp.int32),
            jax.ShapeDtypeStruct((ROWS, 128), F32),
            jax.ShapeDtypeStruct((NTE, 1), jnp.int32),
        ),
    )(eflat, pflat)


def _gemm_body(te_ref, x_ref, prob_ref, wup_ref, wdn_ref, out_ref):
    xb = x_ref[...]
    u = jnp.dot(xb, wup_ref[0], preferred_element_type=F32, precision=jax.lax.Precision.HIGHEST)
    h = u[:, :H]
    g = u[:, H:]
    gelu_h = 0.5 * h * (1.0 + jax.lax.erf(h * 0.7071067811865476))
    act = gelu_h * (g + 1.0)
    act = act * prob_ref[:, :1]
    out_ref[...] = jnp.dot(act, wdn_ref[0], preferred_element_type=F32, precision=jax.lax.Precision.HIGHEST)


def _grouped_gemm(te, x_sorted, prob2d, W_up, W_down):
    grid_spec = pltpu.PrefetchScalarGridSpec(
        num_scalar_prefetch=1,
        grid=(NT,),
        in_specs=[
            pl.BlockSpec((TILE, D), lambda i, te: (i, 0)),
            pl.BlockSpec((TILE, 128), lambda i, te: (i, 0)),
            pl.BlockSpec((1, D, 2 * H), lambda i, te: (te[i], 0, 0)),
            pl.BlockSpec((1, H, D), lambda i, te: (te[i], 0, 0)),
        ],
        out_specs=pl.BlockSpec((TILE, D), lambda i, te: (i, 0)),
    )
    return pl.pallas_call(
        _gemm_body,
        grid_spec=grid_spec,
        out_shape=jax.ShapeDtypeStruct((ROWS, D), F32),
    )(te, x_sorted, prob2d, W_up, W_down)


def kernel(x, expert_p, expert_idxs, W_up, W_down):
    eflat = expert_idxs.astype(jnp.int32).reshape(S, 1)
    pflat = expert_p.astype(F32).reshape(S, 1)
    pos, row_tok, prob2d, te = _routing(eflat, pflat)
    pos = pos.reshape(S)
    row_tok = row_tok.reshape(ROWS)
    te = te.reshape(NTE)

    x_sorted = x[row_tok]
    y_sorted = _grouped_gemm(te, x_sorted, prob2d, W_up, W_down)
    y = y_sorted[pos[0::2]] + y_sorted[pos[1::2]]
    return y
